# baseline (speedup 1.0000x reference)
import functools

import jax
import jax.numpy as jnp
from jax import lax
from jax.experimental import pallas as pl
from jax.experimental.pallas import tpu as pltpu
from jax.experimental.pallas import tpu_sc as plsc

VOCAB = 100000
EMBED_DIM = 64
BATCH = 1024
CTX = 50
MAX_NORM = 1.0

NC, NS = 2, 16
NW = NC * NS
LOOKUPS = BATCH * CTX
PER_W = LOOKUPS // NW
CHUNK = 80
NCHUNK = PER_W // CHUNK


def _sc_gather(x_r, table):
    mesh = plsc.VectorSubcoreMesh(core_axis_name="c", subcore_axis_name="s")

    @functools.partial(
        pl.kernel,
        out_type=jax.ShapeDtypeStruct((NW, NCHUNK, CHUNK, EMBED_DIM),
                                      jnp.float32),
        mesh=mesh,
        compiler_params=pltpu.CompilerParams(use_tc_tiling_on_sc=False),
        scratch_types=[
            pltpu.VMEM((NCHUNK, CHUNK), jnp.int32),
            pltpu.VMEM((NCHUNK, CHUNK, EMBED_DIM), jnp.float32),
            pltpu.SemaphoreType.DMA,
        ],
    )
    def k(x_hbm, table_hbm, out_hbm, idx_v, rows_v, sem):
        wid = lax.axis_index("s") * NC + lax.axis_index("c")
        pltpu.sync_copy(x_hbm.at[wid], idx_v)
        copies = [
            pltpu.async_copy(table_hbm.at[idx_v.at[g]], rows_v.at[g], sem)
            for g in range(NCHUNK)
        ]
        for c in copies:
            c.wait()
        pltpu.sync_copy(rows_v, out_hbm.at[wid])

    return k(x_r, table)


_HCTX = CTX // 2
_B_BLK = 128
_N_POOL = BATCH // _B_BLK
_V_BLK = 2048


def _pool_chunk(e):
    eL, eR = e[:, :EMBED_DIM], e[:, EMBED_DIM:]
    n2L = jnp.sum(eL * eL, axis=1, keepdims=True)
    n2R = jnp.sum(eR * eR, axis=1, keepdims=True)
    sL = eL * jnp.minimum(1.0, MAX_NORM / (jnp.sqrt(n2L) + 1e-7))
    sR = eR * jnp.minimum(1.0, MAX_NORM / (jnp.sqrt(n2R) + 1e-7))
    comb = sL + sR
    r = lax.broadcasted_iota(jnp.int32, (_B_BLK, _B_BLK * _HCTX), 1) // _HCTX
    i = lax.broadcasted_iota(jnp.int32, (_B_BLK, _B_BLK * _HCTX), 0)
    pool = jnp.where(r == i, 1.0 / CTX, 0.0).astype(jnp.float32)
    return lax.dot_general(comb, pool, (((0,), (1,)), ((), ())),
                           preferred_element_type=jnp.float32)


def _mm_body(emb_ref, wt_ref, b_ref, o_ref, ht_s):
    @pl.when(pl.program_id(0) == 0)
    def _():
        for k in range(_N_POOL):
            e = emb_ref[pl.ds(k * _B_BLK * _HCTX, _B_BLK * _HCTX), :]
            ht_s[:, pl.ds(k * _B_BLK, _B_BLK)] = _pool_chunk(e)

    w = wt_ref[...].astype(jnp.bfloat16)
    h = ht_s[...].astype(jnp.bfloat16)
    acc = lax.dot_general(w, h, (((0,), (0,)), ((), ())),
                          preferred_element_type=jnp.float32)
    bcol = jnp.swapaxes(b_ref[...], 0, 1)
    o_ref[...] = acc + bcol


def _matmul(emb_w, wt, brow):
    grid = (pl.cdiv(VOCAB, _V_BLK),)
    return pl.pallas_call(
        _mm_body,
        grid=grid,
        in_specs=[
            pl.BlockSpec((LOOKUPS // 2, 2 * EMBED_DIM), lambda j: (0, 0)),
            pl.BlockSpec((EMBED_DIM, _V_BLK), lambda j: (0, j)),
            pl.BlockSpec((1, _V_BLK), lambda j: (0, j)),
        ],
        out_specs=pl.BlockSpec((_V_BLK, BATCH), lambda j: (j, 0)),
        out_shape=jax.ShapeDtypeStruct((VOCAB, BATCH), jnp.float32),
        scratch_shapes=[pltpu.VMEM((EMBED_DIM, BATCH), jnp.float32)],
    )(emb_w, wt, brow)


def kernel(x, table, W, b):
    x_r = x.reshape(NW, NCHUNK, CHUNK)
    emb_w = _sc_gather(x_r, table).reshape(LOOKUPS // 2, 2 * EMBED_DIM)
    logits_t = _matmul(emb_w, W.T, b.reshape(1, VOCAB))
    return logits_t.T


# speedup vs baseline: 1.0305x; 1.0305x over previous
import functools

import jax
import jax.numpy as jnp
from jax import lax
from jax.experimental import pallas as pl
from jax.experimental.pallas import tpu as pltpu
from jax.experimental.pallas import tpu_sc as plsc

VOCAB = 100000
EMBED_DIM = 64
BATCH = 1024
CTX = 50
MAX_NORM = 1.0

NC, NS = 2, 16
NW = NC * NS
LOOKUPS = BATCH * CTX
PER_W = LOOKUPS // NW
CHUNK = 80
NCHUNK = PER_W // CHUNK


def _sc_gather(x_r, table):
    mesh = plsc.VectorSubcoreMesh(core_axis_name="c", subcore_axis_name="s")

    @functools.partial(
        pl.kernel,
        out_type=jax.ShapeDtypeStruct((NW, NCHUNK, CHUNK, EMBED_DIM),
                                      jnp.float32),
        mesh=mesh,
        compiler_params=pltpu.CompilerParams(use_tc_tiling_on_sc=False),
        scratch_types=[
            pltpu.VMEM((NCHUNK, CHUNK), jnp.int32),
            pltpu.VMEM((NCHUNK, CHUNK, EMBED_DIM), jnp.float32),
            pltpu.SemaphoreType.DMA,
        ],
    )
    def k(x_hbm, table_hbm, out_hbm, idx_v, rows_v, sem):
        wid = lax.axis_index("s") * NC + lax.axis_index("c")
        pltpu.sync_copy(x_hbm.at[wid], idx_v)
        copies = [
            pltpu.async_copy(table_hbm.at[idx_v.at[g]], rows_v.at[g], sem)
            for g in range(NCHUNK)
        ]
        for c in copies:
            c.wait()
        pltpu.sync_copy(rows_v, out_hbm.at[wid])

    return k(x_r, table)


_HCTX = CTX // 2
_B_BLK = 128
_N_POOL = BATCH // _B_BLK
_V_BLK = 4096


def _pool_chunk(e):
    eL, eR = e[:, :EMBED_DIM], e[:, EMBED_DIM:]
    n2L = jnp.sum(eL * eL, axis=1, keepdims=True)
    n2R = jnp.sum(eR * eR, axis=1, keepdims=True)
    sL = eL * jnp.minimum(1.0, MAX_NORM / (jnp.sqrt(n2L) + 1e-7))
    sR = eR * jnp.minimum(1.0, MAX_NORM / (jnp.sqrt(n2R) + 1e-7))
    comb = sL + sR
    r = lax.broadcasted_iota(jnp.int32, (_B_BLK, _B_BLK * _HCTX), 1) // _HCTX
    i = lax.broadcasted_iota(jnp.int32, (_B_BLK, _B_BLK * _HCTX), 0)
    pool = jnp.where(r == i, 1.0 / CTX, 0.0).astype(jnp.float32)
    return lax.dot_general(comb, pool, (((0,), (1,)), ((), ())),
                           preferred_element_type=jnp.float32)


def _pool_body(emb_ref, ht_ref):
    ht_ref[...] = _pool_chunk(emb_ref[...])


def _pool(emb_w):
    grid = (BATCH // _B_BLK,)
    return pl.pallas_call(
        _pool_body,
        grid=grid,
        in_specs=[pl.BlockSpec((_B_BLK * _HCTX, 2 * EMBED_DIM),
                               lambda i: (i, 0))],
        out_specs=pl.BlockSpec((EMBED_DIM, _B_BLK), lambda i: (0, i)),
        out_shape=jax.ShapeDtypeStruct((EMBED_DIM, BATCH), jnp.float32),
    )(emb_w)


def _mm_body(wt_ref, ht_ref, b_ref, o_ref):
    w = wt_ref[...].astype(jnp.bfloat16)
    h = ht_ref[...].astype(jnp.bfloat16)
    acc = lax.dot_general(w, h, (((0,), (0,)), ((), ())),
                          preferred_element_type=jnp.float32)
    bcol = jnp.swapaxes(b_ref[...], 0, 1)
    o_ref[...] = acc + bcol


def _matmul(ht, wt, brow):
    grid = (pl.cdiv(VOCAB, _V_BLK),)
    return pl.pallas_call(
        _mm_body,
        grid=grid,
        in_specs=[
            pl.BlockSpec((EMBED_DIM, _V_BLK), lambda j: (0, j)),
            pl.BlockSpec((EMBED_DIM, BATCH), lambda j: (0, 0)),
            pl.BlockSpec((1, _V_BLK), lambda j: (0, j)),
        ],
        out_specs=pl.BlockSpec((_V_BLK, BATCH), lambda j: (j, 0)),
        out_shape=jax.ShapeDtypeStruct((VOCAB, BATCH), jnp.float32),
    )(wt, ht, brow)


def kernel(x, table, W, b):
    x_r = x.reshape(NW, NCHUNK, CHUNK)
    emb_w = _sc_gather(x_r, table).reshape(LOOKUPS // 2, 2 * EMBED_DIM)
    ht = _pool(emb_w)
    logits_t = _matmul(ht, W.T, b.reshape(1, VOCAB))
    return logits_t.T
